# baseline (speedup 1.0000x reference)
import functools

import jax
import jax.numpy as jnp
from jax import lax
from jax.experimental import pallas as pl
from jax.experimental.pallas import tpu as pltpu
from jax.experimental.pallas import tpu_sc as plsc

B, T, D = 1, 2048, 768
DFF = 3072
E = 8
NE = 2 * T

RBLK = 512
NBS = (NE + E * (RBLK - 1) + RBLK - 1) // RBLK
NPAD = NBS * RBLK

H_TILE = 1024
G_TILE = 512

NW = 32
ECH = NE // NW
CCH = T // NW


def _route_kernel(x_ref, wg_ref, bg_ref, pos_ref, wv_ref, bexp_ref, act_ref):
    y = lax.dot_general(x_ref[...], wg_ref[...], (((1,), (1,)), ((), ())),
                        preferred_element_type=jnp.float32)
    y = y + bg_ref[...]
    idx8 = lax.broadcasted_iota(jnp.int32, (T, E), 1)
    m0 = jnp.max(y, axis=1, keepdims=True)
    i0 = jnp.min(jnp.where(y == m0, idx8, E), axis=1, keepdims=True)
    y2 = jnp.where(idx8 == i0, jnp.float32(-1e30), y)
    m1 = jnp.max(y2, axis=1, keepdims=True)
    i1 = jnp.min(jnp.where(y2 == m1, idx8, E), axis=1, keepdims=True)
    ew = jnp.exp(m1 - m0)
    w0 = 1.0 / (1.0 + ew)
    w1 = 1.0 - w0
    wv_ref[...] = jnp.broadcast_to(jnp.concatenate([w0, w1], axis=0),
                                   (NE, 16))

    oh = jnp.concatenate([(idx8 == i0), (idx8 == i1)], axis=0).astype(jnp.int32)
    inc = oh
    sh = 1
    while sh < NE:
        top = jnp.zeros((sh, E), jnp.int32)
        inc = inc + jnp.concatenate([top, inc[:-sh]], axis=0)
        sh *= 2
    counts = inc[NE - 1:NE]
    rank = inc - oh
    padded = ((counts + RBLK - 1) // RBLK) * RBLK
    t = padded
    sh = 1
    while sh < E:
        left = jnp.zeros((1, sh), jnp.int32)
        t = t + jnp.concatenate([left, t[:, :-sh]], axis=1)
        sh *= 2
    off = t - padded
    pos_ref[...] = jnp.sum((rank + off) * oh, axis=1, keepdims=True)

    ptotal = jnp.sum(padded, axis=1, keepdims=True)
    ii = lax.broadcasted_iota(jnp.int32, (NBS, 1), 0) * RBLK
    slot = jnp.minimum(ii, ptotal - 1)
    ind = jnp.logical_and(slot >= off, slot < off + padded)
    eids = lax.broadcasted_iota(jnp.int32, (NBS, E), 1)
    bexp_ref[...] = jnp.sum(jnp.where(ind, eids, 0), axis=1, keepdims=True)
    act_ref[...] = (ii < ptotal).astype(jnp.int32)


@functools.cache
def _sc_kernels():
    mesh = plsc.VectorSubcoreMesh(core_axis_name="c", subcore_axis_name="s")

    @functools.partial(
        pl.kernel, mesh=mesh,
        out_type=jax.ShapeDtypeStruct((NPAD, D), jnp.float32),
        scratch_types=[pltpu.VMEM((ECH,), jnp.int32),
                       pltpu.VMEM((ECH, D), jnp.float32),
                       pltpu.SemaphoreType.DMA],
    )
    def _dispatch(x_hbm, pos_hbm, xs_hbm, idx_v, rows_v, sem):
        wid = lax.axis_index("s") * 2 + lax.axis_index("c")
        a = wid * ECH
        base = jnp.where(a >= T, a - T, a)
        pltpu.sync_copy(pos_hbm.at[pl.ds(a, ECH)], idx_v)
        pltpu.sync_copy(x_hbm.at[pl.ds(base, ECH)], rows_v)
        pltpu.async_copy(rows_v, xs_hbm.at[idx_v], sem).wait()

    @functools.partial(
        pl.kernel, mesh=mesh,
        out_type=jax.ShapeDtypeStruct((T, D), jnp.float32),
        scratch_types=[pltpu.VMEM((CCH,), jnp.int32),
                       pltpu.VMEM((CCH,), jnp.int32),
                       pltpu.VMEM((CCH, D), jnp.float32),
                       pltpu.VMEM((CCH, D), jnp.float32),
                       pltpu.VMEM((CCH, 16), jnp.float32),
                       pltpu.VMEM((CCH, 16), jnp.float32),
                       pltpu.SemaphoreType.DMA],
    )
    def _combine(ys_hbm, pos_hbm, w_hbm, out_hbm, idx0_v, idx1_v,
                 buf0_v, buf1_v, w0b_v, w1b_v, sem):
        wid = lax.axis_index("s") * 2 + lax.axis_index("c")
        t0 = wid * CCH
        pltpu.sync_copy(pos_hbm.at[pl.ds(t0, CCH)], idx0_v)
        pltpu.sync_copy(pos_hbm.at[pl.ds(T + t0, CCH)], idx1_v)
        pltpu.sync_copy(w_hbm.at[pl.ds(t0, CCH)], w0b_v)
        pltpu.sync_copy(w_hbm.at[pl.ds(T + t0, CCH)], w1b_v)
        pltpu.async_copy(ys_hbm.at[idx0_v], buf0_v, sem).wait()
        pltpu.async_copy(ys_hbm.at[idx1_v], buf1_v, sem).wait()

        def body(r, _):
            wv0 = w0b_v[r, pl.ds(0, 16)]
            wv1 = w1b_v[r, pl.ds(0, 16)]
            for c in range(D // 16):
                sl = pl.ds(c * 16, 16)
                buf0_v[r, sl] = buf0_v[r, sl] * wv0 + buf1_v[r, sl] * wv1
            return 0

        lax.fori_loop(0, CCH, body, 0)
        pltpu.sync_copy(buf0_v, out_hbm.at[pl.ds(t0, CCH)])

    return _dispatch, _combine


def _h_kernel(bexp_ref, act_ref, xs_ref, w1_ref, b1_ref, w2_ref, b2_ref,
              h_ref):
    i = pl.program_id(0)

    @pl.when(act_ref[i] == 1)
    def _():
        xv = xs_ref[...].astype(jnp.bfloat16)
        a = lax.dot_general(xv, w1_ref[0].astype(jnp.bfloat16),
                            (((1,), (1,)), ((), ())),
                            preferred_element_type=jnp.float32) + b1_ref[0]
        b = lax.dot_general(xv, w2_ref[0].astype(jnp.bfloat16),
                            (((1,), (1,)), ((), ())),
                            preferred_element_type=jnp.float32) + b2_ref[0]
        h_ref[...] = a * b


def _out_kernel(bexp_ref, act_ref, h_ref, ws_ref, bs_ref, w3_ref, b3_ref,
                ys_ref):
    i = pl.program_id(0)
    g = pl.program_id(1)
    ng = DFF // G_TILE

    @pl.when(g == 0)
    def _():
        ys_ref[...] = jnp.zeros_like(ys_ref)

    @pl.when(act_ref[i] == 1)
    def _():
        z = lax.dot_general(h_ref[...].astype(jnp.bfloat16),
                            ws_ref[0].astype(jnp.bfloat16),
                            (((1,), (1,)), ((), ())),
                            preferred_element_type=jnp.float32) + bs_ref[0]
        av = (z * jax.nn.sigmoid(z)).astype(jnp.bfloat16)
        ys_ref[...] += lax.dot_general(av, w3_ref[0].astype(jnp.bfloat16),
                                       (((1,), (1,)), ((), ())),
                                       preferred_element_type=jnp.float32)

    @pl.when(jnp.logical_and(g == ng - 1, act_ref[i] == 1))
    def _():
        ys_ref[...] += b3_ref[0]


@jax.jit
def kernel(x, Wg, bg, W1, b1, W2, b2, Ws, bs, W3, b3):
    x2 = x.reshape(T, D)

    pos2, wv2, bexp2, act2 = pl.pallas_call(
        _route_kernel,
        out_shape=[jax.ShapeDtypeStruct((NE, 1), jnp.int32),
                   jax.ShapeDtypeStruct((NE, 16), jnp.float32),
                   jax.ShapeDtypeStruct((NBS, 1), jnp.int32),
                   jax.ShapeDtypeStruct((NBS, 1), jnp.int32)],
    )(x2, Wg, bg.reshape(1, E))
    pos1 = pos2.reshape(NE)
    wv1 = wv2
    bexp = bexp2.reshape(NBS)
    act = act2.reshape(NBS)

    _dispatch, _combine = _sc_kernels()
    xs = _dispatch(x2, pos1)

    nh = DFF // H_TILE
    h = pl.pallas_call(
        _h_kernel,
        grid_spec=pltpu.PrefetchScalarGridSpec(
            num_scalar_prefetch=2,
            grid=(NBS, nh),
            in_specs=[
                pl.BlockSpec((RBLK, D), lambda i, hh, be, ac: (i, 0)),
                pl.BlockSpec((1, H_TILE, D), lambda i, hh, be, ac: (be[i], hh, 0)),
                pl.BlockSpec((1, 1, H_TILE), lambda i, hh, be, ac: (be[i], 0, hh)),
                pl.BlockSpec((1, H_TILE, D), lambda i, hh, be, ac: (be[i], hh, 0)),
                pl.BlockSpec((1, 1, H_TILE), lambda i, hh, be, ac: (be[i], 0, hh)),
            ],
            out_specs=pl.BlockSpec((RBLK, H_TILE), lambda i, hh, be, ac: (i, hh)),
        ),
        out_shape=jax.ShapeDtypeStruct((NPAD, DFF), jnp.float32),
    )(bexp, act, xs, W1, b1.reshape(E, 1, DFF), W2, b2.reshape(E, 1, DFF))

    ng = DFF // G_TILE
    ys = pl.pallas_call(
        _out_kernel,
        grid_spec=pltpu.PrefetchScalarGridSpec(
            num_scalar_prefetch=2,
            grid=(NBS, ng),
            in_specs=[
                pl.BlockSpec((RBLK, DFF), lambda i, g, be, ac: (i, 0)),
                pl.BlockSpec((1, G_TILE, DFF), lambda i, g, be, ac: (be[i], g, 0)),
                pl.BlockSpec((1, 1, G_TILE), lambda i, g, be, ac: (be[i], 0, g)),
                pl.BlockSpec((1, D, G_TILE), lambda i, g, be, ac: (be[i], 0, g)),
                pl.BlockSpec((1, 1, D), lambda i, g, be, ac: (be[i], 0, 0)),
            ],
            out_specs=pl.BlockSpec((RBLK, D), lambda i, g, be, ac: (i, 0)),
        ),
        out_shape=jax.ShapeDtypeStruct((NPAD, D), jnp.float32),
    )(bexp, act, h, Ws, bs.reshape(E, 1, DFF), W3, b3.reshape(E, 1, D))

    out = _combine(ys, pos1, wv1)
    return out.reshape(B, T, D)


# speedup vs baseline: 1.0405x; 1.0405x over previous
import functools

import jax
import jax.numpy as jnp
from jax import lax
from jax.experimental import pallas as pl
from jax.experimental.pallas import tpu as pltpu
from jax.experimental.pallas import tpu_sc as plsc

B, T, D = 1, 2048, 768
DFF = 3072
E = 8
NE = 2 * T

RBLK = 1024
NBS = (NE + E * (RBLK - 1) + RBLK - 1) // RBLK
NPAD = NBS * RBLK

H_TILE = 1024
G_TILE = 512

NW = 32
ECH = NE // NW
CCH = T // NW


def _route_kernel(x_ref, wg_ref, bg_ref, pos_ref, wv_ref, bexp_ref, act_ref):
    y = lax.dot_general(x_ref[...], wg_ref[...], (((1,), (1,)), ((), ())),
                        preferred_element_type=jnp.float32)
    y = y + bg_ref[...]
    idx8 = lax.broadcasted_iota(jnp.int32, (T, E), 1)
    m0 = jnp.max(y, axis=1, keepdims=True)
    i0 = jnp.min(jnp.where(y == m0, idx8, E), axis=1, keepdims=True)
    y2 = jnp.where(idx8 == i0, jnp.float32(-1e30), y)
    m1 = jnp.max(y2, axis=1, keepdims=True)
    i1 = jnp.min(jnp.where(y2 == m1, idx8, E), axis=1, keepdims=True)
    ew = jnp.exp(m1 - m0)
    w0 = 1.0 / (1.0 + ew)
    w1 = 1.0 - w0
    wv_ref[...] = jnp.broadcast_to(jnp.concatenate([w0, w1], axis=0),
                                   (NE, 16))

    oh = jnp.concatenate([(idx8 == i0), (idx8 == i1)], axis=0).astype(jnp.int32)
    inc = oh
    sh = 1
    while sh < NE:
        top = jnp.zeros((sh, E), jnp.int32)
        inc = inc + jnp.concatenate([top, inc[:-sh]], axis=0)
        sh *= 2
    counts = inc[NE - 1:NE]
    rank = inc - oh
    padded = ((counts + RBLK - 1) // RBLK) * RBLK
    t = padded
    sh = 1
    while sh < E:
        left = jnp.zeros((1, sh), jnp.int32)
        t = t + jnp.concatenate([left, t[:, :-sh]], axis=1)
        sh *= 2
    off = t - padded
    pos_ref[...] = jnp.sum((rank + off) * oh, axis=1, keepdims=True)

    ptotal = jnp.sum(padded, axis=1, keepdims=True)
    ii = lax.broadcasted_iota(jnp.int32, (NBS, 1), 0) * RBLK
    slot = jnp.minimum(ii, ptotal - 1)
    ind = jnp.logical_and(slot >= off, slot < off + padded)
    eids = lax.broadcasted_iota(jnp.int32, (NBS, E), 1)
    bexp_ref[...] = jnp.sum(jnp.where(ind, eids, 0), axis=1, keepdims=True)
    act_ref[...] = (ii < ptotal).astype(jnp.int32)


@functools.cache
def _sc_kernels():
    mesh = plsc.VectorSubcoreMesh(core_axis_name="c", subcore_axis_name="s")

    @functools.partial(
        pl.kernel, mesh=mesh,
        out_type=jax.ShapeDtypeStruct((NPAD, D), jnp.float32),
        scratch_types=[pltpu.VMEM((ECH,), jnp.int32),
                       pltpu.VMEM((ECH, D), jnp.float32),
                       pltpu.SemaphoreType.DMA],
    )
    def _dispatch(x_hbm, pos_hbm, xs_hbm, idx_v, rows_v, sem):
        wid = lax.axis_index("s") * 2 + lax.axis_index("c")
        a = wid * ECH
        base = jnp.where(a >= T, a - T, a)
        pltpu.sync_copy(pos_hbm.at[pl.ds(a, ECH)], idx_v)
        pltpu.sync_copy(x_hbm.at[pl.ds(base, ECH)], rows_v)
        pltpu.async_copy(rows_v, xs_hbm.at[idx_v], sem).wait()

    @functools.partial(
        pl.kernel, mesh=mesh,
        out_type=jax.ShapeDtypeStruct((T, D), jnp.float32),
        scratch_types=[pltpu.VMEM((CCH,), jnp.int32),
                       pltpu.VMEM((CCH,), jnp.int32),
                       pltpu.VMEM((CCH, D), jnp.float32),
                       pltpu.VMEM((CCH, D), jnp.float32),
                       pltpu.VMEM((CCH, 16), jnp.float32),
                       pltpu.VMEM((CCH, 16), jnp.float32),
                       pltpu.SemaphoreType.DMA],
    )
    def _combine(ys_hbm, pos_hbm, w_hbm, out_hbm, idx0_v, idx1_v,
                 buf0_v, buf1_v, w0b_v, w1b_v, sem):
        wid = lax.axis_index("s") * 2 + lax.axis_index("c")
        t0 = wid * CCH
        pltpu.sync_copy(pos_hbm.at[pl.ds(t0, CCH)], idx0_v)
        pltpu.sync_copy(pos_hbm.at[pl.ds(T + t0, CCH)], idx1_v)
        pltpu.sync_copy(w_hbm.at[pl.ds(t0, CCH)], w0b_v)
        pltpu.sync_copy(w_hbm.at[pl.ds(T + t0, CCH)], w1b_v)
        pltpu.async_copy(ys_hbm.at[idx0_v], buf0_v, sem).wait()
        pltpu.async_copy(ys_hbm.at[idx1_v], buf1_v, sem).wait()

        def body(r, _):
            wv0 = w0b_v[r, pl.ds(0, 16)]
            wv1 = w1b_v[r, pl.ds(0, 16)]
            for c in range(D // 16):
                sl = pl.ds(c * 16, 16)
                buf0_v[r, sl] = buf0_v[r, sl] * wv0 + buf1_v[r, sl] * wv1
            return 0

        lax.fori_loop(0, CCH, body, 0)
        pltpu.sync_copy(buf0_v, out_hbm.at[pl.ds(t0, CCH)])

    return _dispatch, _combine


NH = DFF // H_TILE
NG = DFF // G_TILE
NS = NH + NG


def _ffn_kernel(bexp_ref, act_ref, xs_ref, w1_ref, b1_ref, w2_ref, b2_ref,
                ws_ref, bs_ref, w3_ref, b3_ref, ys_ref, h_s):
    i = pl.program_id(0)
    s = pl.program_id(1)

    @pl.when(jnp.logical_and(act_ref[i] == 1, s < NH))
    def _():
        xv = xs_ref[...].astype(jnp.bfloat16)
        a = lax.dot_general(xv, w1_ref[0].astype(jnp.bfloat16),
                            (((1,), (1,)), ((), ())),
                            preferred_element_type=jnp.float32) + b1_ref[0]
        b = lax.dot_general(xv, w2_ref[0].astype(jnp.bfloat16),
                            (((1,), (1,)), ((), ())),
                            preferred_element_type=jnp.float32) + b2_ref[0]
        h_s[:, pl.ds(s * H_TILE, H_TILE)] = (a * b).astype(jnp.bfloat16)

    @pl.when(jnp.logical_and(act_ref[i] == 1, s >= NH))
    def _():
        z = lax.dot_general(h_s[...], ws_ref[0].astype(jnp.bfloat16),
                            (((1,), (1,)), ((), ())),
                            preferred_element_type=jnp.float32) + bs_ref[0]
        av = (z * jax.nn.sigmoid(z)).astype(jnp.bfloat16)
        part = lax.dot_general(av, w3_ref[0].astype(jnp.bfloat16),
                               (((1,), (1,)), ((), ())),
                               preferred_element_type=jnp.float32)

        @pl.when(s == NH)
        def _():
            ys_ref[...] = part

        @pl.when(s > NH)
        def _():
            ys_ref[...] += part

        @pl.when(s == NS - 1)
        def _():
            ys_ref[...] += b3_ref[0]


@jax.jit
def kernel(x, Wg, bg, W1, b1, W2, b2, Ws, bs, W3, b3):
    x2 = x.reshape(T, D)

    pos2, wv2, bexp2, act2 = pl.pallas_call(
        _route_kernel,
        out_shape=[jax.ShapeDtypeStruct((NE, 1), jnp.int32),
                   jax.ShapeDtypeStruct((NE, 16), jnp.float32),
                   jax.ShapeDtypeStruct((NBS, 1), jnp.int32),
                   jax.ShapeDtypeStruct((NBS, 1), jnp.int32)],
    )(x2, Wg, bg.reshape(1, E))
    pos1 = pos2.reshape(NE)
    wv1 = wv2
    bexp = bexp2.reshape(NBS)
    act = act2.reshape(NBS)

    _dispatch, _combine = _sc_kernels()
    xs = _dispatch(x2, pos1)

    ys = pl.pallas_call(
        _ffn_kernel,
        grid_spec=pltpu.PrefetchScalarGridSpec(
            num_scalar_prefetch=2,
            grid=(NBS, NS),
            in_specs=[
                pl.BlockSpec((RBLK, D), lambda i, s, be, ac: (i, 0)),
                pl.BlockSpec((1, H_TILE, D),
                             lambda i, s, be, ac:
                             (be[i], jnp.minimum(s, NH - 1), 0)),
                pl.BlockSpec((1, 1, H_TILE),
                             lambda i, s, be, ac:
                             (be[i], 0, jnp.minimum(s, NH - 1))),
                pl.BlockSpec((1, H_TILE, D),
                             lambda i, s, be, ac:
                             (be[i], jnp.minimum(s, NH - 1), 0)),
                pl.BlockSpec((1, 1, H_TILE),
                             lambda i, s, be, ac:
                             (be[i], 0, jnp.minimum(s, NH - 1))),
                pl.BlockSpec((1, G_TILE, DFF),
                             lambda i, s, be, ac:
                             (be[i], jnp.maximum(s - NH, 0), 0)),
                pl.BlockSpec((1, 1, G_TILE),
                             lambda i, s, be, ac:
                             (be[i], 0, jnp.maximum(s - NH, 0))),
                pl.BlockSpec((1, D, G_TILE),
                             lambda i, s, be, ac:
                             (be[i], 0, jnp.maximum(s - NH, 0))),
                pl.BlockSpec((1, 1, D), lambda i, s, be, ac: (be[i], 0, 0)),
            ],
            out_specs=pl.BlockSpec((RBLK, D), lambda i, s, be, ac: (i, 0)),
            scratch_shapes=[pltpu.VMEM((RBLK, DFF), jnp.bfloat16)],
        ),
        out_shape=jax.ShapeDtypeStruct((NPAD, D), jnp.float32),
    )(bexp, act, xs, W1, b1.reshape(E, 1, DFF), W2, b2.reshape(E, 1, DFF),
      Ws, bs.reshape(E, 1, DFF), W3, b3.reshape(E, 1, D))

    out = _combine(ys, pos1, wv1)
    return out.reshape(B, T, D)


# speedup vs baseline: 1.0944x; 1.0518x over previous
import functools

import jax
import jax.numpy as jnp
from jax import lax
from jax.experimental import pallas as pl
from jax.experimental.pallas import tpu as pltpu
from jax.experimental.pallas import tpu_sc as plsc

B, T, D = 1, 2048, 768
DFF = 3072
E = 8
NE = 2 * T

RBLK = 576
NBS = (NE + E * (RBLK - 1) + RBLK - 1) // RBLK
NPAD = NBS * RBLK

H_TILE = 1024
G_TILE = 512

NW = 32
ECH = NE // NW
CCH = T // NW


def _route_kernel(x_ref, wg_ref, bg_ref, pos_ref, wv_ref, bexp_ref, act_ref):
    y = lax.dot_general(x_ref[...], wg_ref[...], (((1,), (1,)), ((), ())),
                        preferred_element_type=jnp.float32)
    y = y + bg_ref[...]
    idx8 = lax.broadcasted_iota(jnp.int32, (T, E), 1)
    m0 = jnp.max(y, axis=1, keepdims=True)
    i0 = jnp.min(jnp.where(y == m0, idx8, E), axis=1, keepdims=True)
    y2 = jnp.where(idx8 == i0, jnp.float32(-1e30), y)
    m1 = jnp.max(y2, axis=1, keepdims=True)
    i1 = jnp.min(jnp.where(y2 == m1, idx8, E), axis=1, keepdims=True)
    ew = jnp.exp(m1 - m0)
    w0 = 1.0 / (1.0 + ew)
    w1 = 1.0 - w0
    wv_ref[...] = jnp.broadcast_to(jnp.concatenate([w0, w1], axis=0),
                                   (NE, 16))

    oh = jnp.concatenate([(idx8 == i0), (idx8 == i1)], axis=0).astype(jnp.int32)
    inc = oh
    sh = 1
    while sh < NE:
        top = jnp.zeros((sh, E), jnp.int32)
        inc = inc + jnp.concatenate([top, inc[:-sh]], axis=0)
        sh *= 2
    counts = inc[NE - 1:NE]
    rank = inc - oh
    padded = ((counts + RBLK - 1) // RBLK) * RBLK
    t = padded
    sh = 1
    while sh < E:
        left = jnp.zeros((1, sh), jnp.int32)
        t = t + jnp.concatenate([left, t[:, :-sh]], axis=1)
        sh *= 2
    off = t - padded
    pos_ref[...] = jnp.sum((rank + off) * oh, axis=1, keepdims=True)

    ptotal = jnp.sum(padded, axis=1, keepdims=True)
    ii = lax.broadcasted_iota(jnp.int32, (NBS, 1), 0) * RBLK
    slot = jnp.minimum(ii, ptotal - 1)
    ind = jnp.logical_and(slot >= off, slot < off + padded)
    eids = lax.broadcasted_iota(jnp.int32, (NBS, E), 1)
    bexp_ref[...] = jnp.sum(jnp.where(ind, eids, 0), axis=1, keepdims=True)
    act_ref[...] = (ii < ptotal).astype(jnp.int32)


@functools.cache
def _sc_kernels():
    mesh = plsc.VectorSubcoreMesh(core_axis_name="c", subcore_axis_name="s")

    @functools.partial(
        pl.kernel, mesh=mesh,
        out_type=jax.ShapeDtypeStruct((NPAD, D), jnp.float32),
        scratch_types=[pltpu.VMEM((ECH,), jnp.int32),
                       pltpu.VMEM((ECH, D), jnp.float32),
                       pltpu.SemaphoreType.DMA],
    )
    def _dispatch(x_hbm, pos_hbm, xs_hbm, idx_v, rows_v, sem):
        wid = lax.axis_index("s") * 2 + lax.axis_index("c")
        a = wid * ECH
        base = jnp.where(a >= T, a - T, a)
        pltpu.sync_copy(pos_hbm.at[pl.ds(a, ECH)], idx_v)
        pltpu.sync_copy(x_hbm.at[pl.ds(base, ECH)], rows_v)
        pltpu.async_copy(rows_v, xs_hbm.at[idx_v], sem).wait()

    @functools.partial(
        pl.kernel, mesh=mesh,
        out_type=jax.ShapeDtypeStruct((T, D), jnp.float32),
        scratch_types=[pltpu.VMEM((CCH,), jnp.int32),
                       pltpu.VMEM((CCH,), jnp.int32),
                       pltpu.VMEM((CCH, D), jnp.float32),
                       pltpu.VMEM((CCH, D), jnp.float32),
                       pltpu.VMEM((CCH, 16), jnp.float32),
                       pltpu.VMEM((CCH, 16), jnp.float32),
                       pltpu.SemaphoreType.DMA],
    )
    def _combine(ys_hbm, pos_hbm, w_hbm, out_hbm, idx0_v, idx1_v,
                 buf0_v, buf1_v, w0b_v, w1b_v, sem):
        wid = lax.axis_index("s") * 2 + lax.axis_index("c")
        t0 = wid * CCH
        pltpu.sync_copy(pos_hbm.at[pl.ds(t0, CCH)], idx0_v)
        pltpu.sync_copy(pos_hbm.at[pl.ds(T + t0, CCH)], idx1_v)
        pltpu.sync_copy(w_hbm.at[pl.ds(t0, CCH)], w0b_v)
        pltpu.sync_copy(w_hbm.at[pl.ds(T + t0, CCH)], w1b_v)
        pltpu.async_copy(ys_hbm.at[idx0_v], buf0_v, sem).wait()
        pltpu.async_copy(ys_hbm.at[idx1_v], buf1_v, sem).wait()

        def body(r, _):
            wv0 = w0b_v[r, pl.ds(0, 16)]
            wv1 = w1b_v[r, pl.ds(0, 16)]
            for c in range(D // 16):
                sl = pl.ds(c * 16, 16)
                buf0_v[r, sl] = buf0_v[r, sl] * wv0 + buf1_v[r, sl] * wv1
            return 0

        lax.fori_loop(0, CCH, body, 0)
        pltpu.sync_copy(buf0_v, out_hbm.at[pl.ds(t0, CCH)])

    return _dispatch, _combine


NH = DFF // H_TILE
NG = DFF // G_TILE
NS = NH + NG


def _ffn_kernel(bexp_ref, act_ref, xs_ref, w1_ref, b1_ref, w2_ref, b2_ref,
                ws_ref, bs_ref, w3_ref, b3_ref, ys_ref, h_s):
    i = pl.program_id(0)
    s = pl.program_id(1)

    @pl.when(jnp.logical_and(act_ref[i] == 1, s < NH))
    def _():
        xv = xs_ref[...].astype(jnp.bfloat16)
        a = lax.dot_general(xv, w1_ref[0].astype(jnp.bfloat16),
                            (((1,), (1,)), ((), ())),
                            preferred_element_type=jnp.float32) + b1_ref[0]
        b = lax.dot_general(xv, w2_ref[0].astype(jnp.bfloat16),
                            (((1,), (1,)), ((), ())),
                            preferred_element_type=jnp.float32) + b2_ref[0]
        h_s[:, pl.ds(s * H_TILE, H_TILE)] = (a * b).astype(jnp.bfloat16)

    @pl.when(jnp.logical_and(act_ref[i] == 1, s >= NH))
    def _():
        z = lax.dot_general(h_s[...], ws_ref[0].astype(jnp.bfloat16),
                            (((1,), (1,)), ((), ())),
                            preferred_element_type=jnp.float32) + bs_ref[0]
        av = (z * jax.nn.sigmoid(z)).astype(jnp.bfloat16)
        part = lax.dot_general(av, w3_ref[0].astype(jnp.bfloat16),
                               (((1,), (1,)), ((), ())),
                               preferred_element_type=jnp.float32)

        @pl.when(s == NH)
        def _():
            ys_ref[...] = part

        @pl.when(s > NH)
        def _():
            ys_ref[...] += part

        @pl.when(s == NS - 1)
        def _():
            ys_ref[...] += b3_ref[0]


@jax.jit
def kernel(x, Wg, bg, W1, b1, W2, b2, Ws, bs, W3, b3):
    x2 = x.reshape(T, D)

    pos2, wv2, bexp2, act2 = pl.pallas_call(
        _route_kernel,
        out_shape=[jax.ShapeDtypeStruct((NE, 1), jnp.int32),
                   jax.ShapeDtypeStruct((NE, 16), jnp.float32),
                   jax.ShapeDtypeStruct((NBS, 1), jnp.int32),
                   jax.ShapeDtypeStruct((NBS, 1), jnp.int32)],
    )(x2, Wg, bg.reshape(1, E))
    pos1 = pos2.reshape(NE)
    wv1 = wv2
    bexp = bexp2.reshape(NBS)
    act = act2.reshape(NBS)

    _dispatch, _combine = _sc_kernels()
    xs = _dispatch(x2, pos1)

    ys = pl.pallas_call(
        _ffn_kernel,
        grid_spec=pltpu.PrefetchScalarGridSpec(
            num_scalar_prefetch=2,
            grid=(NBS, NS),
            in_specs=[
                pl.BlockSpec((RBLK, D), lambda i, s, be, ac: (i, 0)),
                pl.BlockSpec((1, H_TILE, D),
                             lambda i, s, be, ac:
                             (be[i], jnp.minimum(s, NH - 1), 0)),
                pl.BlockSpec((1, 1, H_TILE),
                             lambda i, s, be, ac:
                             (be[i], 0, jnp.minimum(s, NH - 1))),
                pl.BlockSpec((1, H_TILE, D),
                             lambda i, s, be, ac:
                             (be[i], jnp.minimum(s, NH - 1), 0)),
                pl.BlockSpec((1, 1, H_TILE),
                             lambda i, s, be, ac:
                             (be[i], 0, jnp.minimum(s, NH - 1))),
                pl.BlockSpec((1, G_TILE, DFF),
                             lambda i, s, be, ac:
                             (be[i], jnp.maximum(s - NH, 0), 0)),
                pl.BlockSpec((1, 1, G_TILE),
                             lambda i, s, be, ac:
                             (be[i], 0, jnp.maximum(s - NH, 0))),
                pl.BlockSpec((1, D, G_TILE),
                             lambda i, s, be, ac:
                             (be[i], 0, jnp.maximum(s - NH, 0))),
                pl.BlockSpec((1, 1, D), lambda i, s, be, ac: (be[i], 0, 0)),
            ],
            out_specs=pl.BlockSpec((RBLK, D), lambda i, s, be, ac: (i, 0)),
            scratch_shapes=[pltpu.VMEM((RBLK, DFF), jnp.bfloat16)],
        ),
        out_shape=jax.ShapeDtypeStruct((NPAD, D), jnp.float32),
    )(bexp, act, xs, W1, b1.reshape(E, 1, DFF), W2, b2.reshape(E, 1, DFF),
      Ws, bs.reshape(E, 1, DFF), W3, b3.reshape(E, 1, D))

    out = _combine(ys, pos1, wv1)
    return out.reshape(B, T, D)
